# baseline (speedup 1.0000x reference)
import functools

import jax
import jax.numpy as jnp
from jax import lax
from jax.experimental import pallas as pl
from jax.experimental.pallas import tpu as pltpu
from jax.experimental.pallas import tpu_sc as plsc

_B = 512
_D = 64
_L = 16


def _sqrt16(x):
    i = plsc.bitcast(x, jnp.int32)
    i = jnp.int32(0x5F3759DF) - lax.shift_right_logical(i, jnp.full((_L,), 1, jnp.int32))
    y = plsc.bitcast(i, jnp.float32)
    for _ in range(3):
        y = y * (1.5 - 0.5 * x * y * y)
    return jnp.where(x > 0.0, x * y, 0.0)


def _make_kernel():
    nw = 16
    rows_per = _B // nw
    mesh = plsc.VectorSubcoreMesh(
        core_axis_name="c", subcore_axis_name="s", num_cores=1)

    @functools.partial(
        pl.kernel,
        mesh=mesh,
        out_type=jax.ShapeDtypeStruct((_B,), jnp.float32),
        compiler_params=pltpu.CompilerParams(needs_layout_passes=False),
        scratch_types=[
            pltpu.VMEM((rows_per,), jnp.int32),
            pltpu.VMEM((rows_per,), jnp.int32),
            pltpu.VMEM((rows_per, 2 * _D), jnp.float32),
            pltpu.VMEM((rows_per, 2 * _D), jnp.float32),
            pltpu.VMEM((rows_per,), jnp.float32),
            pltpu.SemaphoreType.DMA,
            pltpu.SemaphoreType.DMA,
        ],
    )
    def k(cidx_hbm, didx_hbm, table_hbm, out_hbm,
          cidx_v, didx_v, crows_v, drows_v, res_v, sem_c, sem_d):
        base = lax.axis_index("s") * rows_per

        ip_c = pltpu.async_copy(cidx_hbm.at[pl.ds(base, rows_per)], cidx_v, sem_c)
        ip_d = pltpu.async_copy(didx_hbm.at[pl.ds(base, rows_per)], didx_v, sem_d)
        ip_c.wait()
        ip_d.wait()

        cp_c = pltpu.async_copy(table_hbm.at[cidx_v], crows_v, sem_c)
        cp_d = pltpu.async_copy(table_hbm.at[didx_v], drows_v, sem_d)
        cp_c.wait()
        cp_d.wait()

        zeros = jnp.zeros((_L,), jnp.int32)
        z = jnp.zeros((_L,), jnp.float32)
        _U = 8

        for g in range(rows_per // _L):
            rows_i = lax.iota(jnp.int32, _L) + g * _L

            def body(j, carry):
                ssq_lb, ssq_rt, shp = carry
                for u in range(_U):
                    col = zeros + (j * _U + u)
                    c1 = plsc.load_gather(crows_v, [rows_i, col])
                    c2 = plsc.load_gather(crows_v, [rows_i, col + _D])
                    d1 = plsc.load_gather(drows_v, [rows_i, col])
                    d2 = plsc.load_gather(drows_v, [rows_i, col + _D])
                    t1 = jnp.maximum(d1 - c1, 0.0)
                    t2 = jnp.maximum(c1 + c2 - d2 - d1, 0.0)
                    ssq_lb = ssq_lb + t1 * t1
                    ssq_rt = ssq_rt + t2 * t2
                    shp = shp + jnp.maximum(-c2, 0.0) + jnp.maximum(-d2, 0.0)
                return ssq_lb, ssq_rt, shp

            ssq_lb, ssq_rt, shp = lax.fori_loop(0, _D // _U, body, (z, z, z))
            res_v[pl.ds(g * _L, _L)] = _sqrt16(ssq_lb) + _sqrt16(ssq_rt) + shp

        pltpu.sync_copy(res_v, out_hbm.at[pl.ds(base, rows_per)])

    return k


_sc_kernel = jax.jit(_make_kernel())


def kernel(input, class_emb):
    batch = input[:_B]
    out = _sc_kernel(batch[:, 0], batch[:, 1], class_emb)
    return out.reshape(_B, 1)


# speedup vs baseline: 1.0235x; 1.0235x over previous
import functools

import jax
import jax.numpy as jnp
from jax import lax
from jax.experimental import pallas as pl
from jax.experimental.pallas import tpu as pltpu
from jax.experimental.pallas import tpu_sc as plsc

_B = 512
_D = 64
_L = 16


def _sqrt16(x):
    i = plsc.bitcast(x, jnp.int32)
    i = jnp.int32(0x5F3759DF) - lax.shift_right_logical(i, jnp.full((_L,), 1, jnp.int32))
    y = plsc.bitcast(i, jnp.float32)
    for _ in range(3):
        y = y * (1.5 - 0.5 * x * y * y)
    return jnp.where(x > 0.0, x * y, 0.0)


def _make_kernel():
    info = plsc.get_sparse_core_info()
    nc, ns = info.num_cores, info.num_subcores
    nw = nc * ns
    rows_per = _B // nw
    mesh = plsc.VectorSubcoreMesh(core_axis_name="c", subcore_axis_name="s")

    @functools.partial(
        pl.kernel,
        mesh=mesh,
        out_type=jax.ShapeDtypeStruct((_B,), jnp.float32),
        compiler_params=pltpu.CompilerParams(
            needs_layout_passes=False,
            skip_device_barrier=True,
        ),
        scratch_types=[
            pltpu.VMEM((rows_per,), jnp.int32),
            pltpu.VMEM((rows_per,), jnp.int32),
            pltpu.VMEM((rows_per, 2 * _D), jnp.float32),
            pltpu.VMEM((rows_per, 2 * _D), jnp.float32),
            pltpu.VMEM((rows_per,), jnp.float32),
            pltpu.SemaphoreType.DMA,
            pltpu.SemaphoreType.DMA,
        ],
    )
    def k(cidx_hbm, didx_hbm, table_hbm, out_hbm,
          cidx_v, didx_v, crows_v, drows_v, res_v, sem_c, sem_d):
        wid = lax.axis_index("s") * nc + lax.axis_index("c")
        base = wid * rows_per

        ip_c = pltpu.async_copy(cidx_hbm.at[pl.ds(base, rows_per)], cidx_v, sem_c)
        ip_d = pltpu.async_copy(didx_hbm.at[pl.ds(base, rows_per)], didx_v, sem_d)
        rows_i = lax.iota(jnp.int32, _L)
        zeros = jnp.zeros((_L,), jnp.int32)
        ip_c.wait()
        cp_c = pltpu.async_copy(table_hbm.at[cidx_v], crows_v, sem_c)
        ip_d.wait()
        cp_d = pltpu.async_copy(table_hbm.at[didx_v], drows_v, sem_d)

        z = jnp.zeros((_L,), jnp.float32)
        _U = 8

        cp_c.wait()

        def body_c(j, carry):
            shp = carry
            for u in range(_U):
                col = zeros + (j * _U + u)
                c2 = plsc.load_gather(crows_v, [rows_i, col + _D])
                shp = shp + jnp.maximum(-c2, 0.0)
            return shp

        shp_c = lax.fori_loop(0, _D // _U, body_c, z)
        cp_d.wait()

        def body(j, carry):
            ssq_lb, ssq_rt, shp = carry
            for u in range(_U):
                col = zeros + (j * _U + u)
                c1 = plsc.load_gather(crows_v, [rows_i, col])
                c2 = plsc.load_gather(crows_v, [rows_i, col + _D])
                d1 = plsc.load_gather(drows_v, [rows_i, col])
                d2 = plsc.load_gather(drows_v, [rows_i, col + _D])
                t1 = jnp.maximum(d1 - c1, 0.0)
                t2 = jnp.maximum(c1 + c2 - d2 - d1, 0.0)
                ssq_lb = ssq_lb + t1 * t1
                ssq_rt = ssq_rt + t2 * t2
                shp = shp + jnp.maximum(-d2, 0.0)
            return ssq_lb, ssq_rt, shp

        ssq_lb, ssq_rt, shp = lax.fori_loop(0, _D // _U, body, (z, z, shp_c))

        res_v[...] = _sqrt16(ssq_lb) + _sqrt16(ssq_rt) + shp
        pltpu.sync_copy(res_v, out_hbm.at[pl.ds(base, rows_per)])

    return k


_sc_kernel = jax.jit(_make_kernel())


def kernel(input, class_emb):
    batch = input[:_B]
    out = _sc_kernel(batch[:, 0], batch[:, 1], class_emb)
    return out.reshape(_B, 1)


# speedup vs baseline: 1.0357x; 1.0120x over previous
import functools

import jax
import jax.numpy as jnp
from jax import lax
from jax.experimental import pallas as pl
from jax.experimental.pallas import tpu as pltpu
from jax.experimental.pallas import tpu_sc as plsc

_B = 512
_D = 64
_L = 16


def _sqrt16(x):
    i = plsc.bitcast(x, jnp.int32)
    i = jnp.int32(0x5F3759DF) - lax.shift_right_logical(i, jnp.full((_L,), 1, jnp.int32))
    y = plsc.bitcast(i, jnp.float32)
    for _ in range(3):
        y = y * (1.5 - 0.5 * x * y * y)
    return jnp.where(x > 0.0, x * y, 0.0)


def _make_kernel():
    info = plsc.get_sparse_core_info()
    nc, ns = info.num_cores, info.num_subcores
    nw = nc * ns
    rows_per = _B // nw
    mesh = plsc.VectorSubcoreMesh(core_axis_name="c", subcore_axis_name="s")

    @functools.partial(
        pl.kernel,
        mesh=mesh,
        out_type=jax.ShapeDtypeStruct((_B,), jnp.float32),
        compiler_params=pltpu.CompilerParams(
            needs_layout_passes=False,
            skip_device_barrier=True,
        ),
        scratch_types=[
            pltpu.VMEM((rows_per,), jnp.int32),
            pltpu.VMEM((rows_per,), jnp.int32),
            pltpu.VMEM((rows_per, 2 * _D), jnp.float32),
            pltpu.VMEM((rows_per, 2 * _D), jnp.float32),
            pltpu.VMEM((rows_per,), jnp.float32),
            pltpu.SemaphoreType.DMA,
            pltpu.SemaphoreType.DMA,
        ],
    )
    def k(cidx_hbm, didx_hbm, table_hbm, out_hbm,
          cidx_v, didx_v, crows_v, drows_v, res_v, sem_c, sem_d):
        wid = lax.axis_index("s") * nc + lax.axis_index("c")
        base = wid * rows_per

        ip_c = pltpu.async_copy(cidx_hbm.at[pl.ds(base, rows_per)], cidx_v, sem_c)
        ip_d = pltpu.async_copy(didx_hbm.at[pl.ds(base, rows_per)], didx_v, sem_d)
        rows_i = lax.iota(jnp.int32, _L)
        zeros = jnp.zeros((_L,), jnp.int32)
        ip_c.wait()
        ip_d.wait()

        cp_c = pltpu.async_copy(table_hbm.at[cidx_v], crows_v, sem_c)
        cp_d = pltpu.async_copy(table_hbm.at[didx_v], drows_v, sem_d)
        cp_c.wait()
        cp_d.wait()

        z = jnp.zeros((_L,), jnp.float32)
        _U = 8

        def body(j, carry):
            ssq_lb, ssq_rt, shp = carry
            for u in range(_U):
                col = zeros + (j * _U + u)
                c1 = plsc.load_gather(crows_v, [rows_i, col])
                c2 = plsc.load_gather(crows_v, [rows_i, col + _D])
                d1 = plsc.load_gather(drows_v, [rows_i, col])
                d2 = plsc.load_gather(drows_v, [rows_i, col + _D])
                t1 = jnp.maximum(d1 - c1, 0.0)
                t2 = jnp.maximum(c1 + c2 - d2 - d1, 0.0)
                ssq_lb = ssq_lb + t1 * t1
                ssq_rt = ssq_rt + t2 * t2
                shp = shp + jnp.maximum(-c2, 0.0) + jnp.maximum(-d2, 0.0)
            return ssq_lb, ssq_rt, shp

        ssq_lb, ssq_rt, shp = lax.fori_loop(0, _D // _U, body, (z, z, z))

        res_v[...] = _sqrt16(ssq_lb) + _sqrt16(ssq_rt) + shp
        pltpu.sync_copy(res_v, out_hbm.at[pl.ds(base, rows_per)])

    return k


_sc_kernel = jax.jit(_make_kernel())


def kernel(input, class_emb):
    batch = input[:_B]
    out = _sc_kernel(batch[:, 0], batch[:, 1], class_emb)
    return out.reshape(_B, 1)
